# baseline (speedup 1.0000x reference)
import functools

import jax
import jax.numpy as jnp
from jax import lax
from jax.experimental import pallas as pl
from jax.experimental.pallas import tpu as pltpu
from jax.experimental.pallas import tpu_sc as plsc

C = 16
L = 16


def kernel(x, pos, pos_embedding):
    B, S, D = x.shape
    N = B * S
    xf = x.reshape(N, D)
    idx = pos.reshape(N).astype(jnp.int32)

    info = plsc.get_sparse_core_info()
    NC, NS = info.num_cores, info.num_subcores
    NW = NC * NS
    R = N // NW
    n_chunks = R // C

    mesh = plsc.VectorSubcoreMesh(core_axis_name="core", subcore_axis_name="subcore")

    @functools.partial(
        pl.kernel,
        out_type=jax.ShapeDtypeStruct((N, D), x.dtype),
        mesh=mesh,
        scratch_types=[
            pltpu.VMEM((R,), jnp.int32),
            pltpu.VMEM((C, D), jnp.float32),
            pltpu.VMEM((C, D), jnp.float32),
        ],
    )
    def run(x_hbm, i_hbm, t_hbm, o_hbm, idx_v, buf_g, buf_x):
        wid = lax.axis_index("core") * NS + lax.axis_index("subcore")
        base = wid * R
        pltpu.sync_copy(i_hbm.at[pl.ds(base, R)], idx_v)

        @pl.loop(0, n_chunks)
        def _(c):
            row0 = base + c * C
            pltpu.sync_copy(t_hbm.at[idx_v.at[pl.ds(c * C, C)]], buf_g)
            pltpu.sync_copy(x_hbm.at[pl.ds(row0, C)], buf_x)

            @pl.loop(0, C)
            def _(r):
                for c0 in range(0, D, L):
                    s = (r, pl.ds(c0, L))
                    buf_g.at[s][...] += buf_x.at[s][...]

            pltpu.sync_copy(buf_g, o_hbm.at[pl.ds(row0, C)])

    out = run(xf, idx, pos_embedding)
    return out.reshape(B, S, D)


# speedup vs baseline: 1.9565x; 1.9565x over previous
import functools

import jax
import jax.numpy as jnp
from jax import lax
from jax.experimental import pallas as pl
from jax.experimental.pallas import tpu as pltpu
from jax.experimental.pallas import tpu_sc as plsc

C = 16
L = 16


def kernel(x, pos, pos_embedding):
    B, S, D = x.shape
    N = B * S
    xf = x.reshape(N, D)
    idx = pos.reshape(N).astype(jnp.int32)

    info = plsc.get_sparse_core_info()
    NC, NS = info.num_cores, info.num_subcores
    NW = NC * NS
    R = N // NW
    n_chunks = R // C

    mesh = plsc.VectorSubcoreMesh(core_axis_name="core", subcore_axis_name="subcore")

    @functools.partial(
        pl.kernel,
        out_type=jax.ShapeDtypeStruct((N, D), x.dtype),
        mesh=mesh,
        scratch_types=[
            pltpu.VMEM((R,), jnp.int32),
            pltpu.VMEM((C, D), jnp.float32),
            pltpu.VMEM((C, D), jnp.float32),
            pltpu.VMEM((C, D), jnp.float32),
            pltpu.VMEM((C, D), jnp.float32),
            pltpu.SemaphoreType.DMA,
            pltpu.SemaphoreType.DMA,
            pltpu.SemaphoreType.DMA,
            pltpu.SemaphoreType.DMA,
            pltpu.SemaphoreType.DMA,
            pltpu.SemaphoreType.DMA,
        ],
    )
    def run(x_hbm, i_hbm, t_hbm, o_hbm, idx_v,
            bg0, bg1, bx0, bx1, sg0, sg1, sx0, sx1, so0, so1):
        bg = (bg0, bg1)
        bx = (bx0, bx1)
        sg = (sg0, sg1)
        sx = (sx0, sx1)
        so = (so0, so1)

        wid = lax.axis_index("core") * NS + lax.axis_index("subcore")
        base = wid * R
        pltpu.sync_copy(i_hbm.at[pl.ds(base, R)], idx_v)

        def start_in(c, b):
            pltpu.async_copy(t_hbm.at[idx_v.at[pl.ds(c * C, C)]], bg[b], sg[b])
            pltpu.async_copy(x_hbm.at[pl.ds(base + c * C, C)], bx[b], sx[b])

        def wait_in(c, b):
            pltpu.make_async_copy(
                t_hbm.at[idx_v.at[pl.ds(c * C, C)]], bg[b], sg[b]).wait()
            pltpu.make_async_copy(
                x_hbm.at[pl.ds(base + c * C, C)], bx[b], sx[b]).wait()

        def wait_out(b):
            pltpu.make_async_copy(bg[b], o_hbm.at[pl.ds(base, C)], so[b]).wait()

        start_in(0, 0)

        @pl.loop(0, n_chunks // 2)
        def _(p):
            for b in range(2):
                c = p * 2 + b

                @pl.when(c + 1 < n_chunks)
                def _():
                    @pl.when(c + 1 >= 2)
                    def _():
                        wait_out(1 - b)

                    start_in(c + 1, 1 - b)

                wait_in(c, b)

                @pl.loop(0, C)
                def _(r):
                    for c0 in range(0, D, L):
                        s = (r, pl.ds(c0, L))
                        bg[b].at[s][...] += bx[b].at[s][...]

                pltpu.async_copy(bg[b], o_hbm.at[pl.ds(base + c * C, C)], so[b])

        wait_out(0)
        wait_out(1)

    out = run(xf, idx, pos_embedding)
    return out.reshape(B, S, D)
